# baseline (speedup 1.0000x reference)
import functools

import numpy as np

import jax
import jax.numpy as jnp
from jax.experimental import pallas as pl
from jax.experimental.pallas import tpu as pltpu

_LANES = 128


def _gem_body(p_ref, x_ref, w_ref, o_ref, *, hw, eps):
    p = p_ref[0, 0]
    xc = jnp.maximum(x_ref[...], eps)
    xp = jnp.exp(jnp.log(xc) * p)
    s = jax.lax.dot_general(
        xp.astype(jnp.bfloat16), w_ref[...],
        (((1,), (0,)), ((), ())), preferred_element_type=jnp.float32)
    m = s * (1.0 / hw)
    o_ref[...] = jnp.exp(jnp.log(m) / p)


def kernel(x, p):
    N, C, H, W = x.shape
    HW = H * W
    NC = N * C
    R = NC // _LANES
    ROWW = HW * _LANES
    x2 = x.reshape(R, ROWW)

    e = np.arange(ROWW)
    w = jnp.asarray(e[:, None] // HW == np.arange(_LANES)[None, :],
                    jnp.bfloat16)

    p_arr = jnp.asarray(p, jnp.float32).reshape(1, 1)

    TB = 256
    while R % TB:
        TB //= 2

    out = pl.pallas_call(
        functools.partial(_gem_body, hw=HW, eps=1e-6),
        out_shape=jax.ShapeDtypeStruct((R, _LANES), jnp.float32),
        grid=(R // TB,),
        in_specs=[
            pl.BlockSpec(memory_space=pltpu.MemorySpace.SMEM),
            pl.BlockSpec((TB, ROWW), lambda i: (i, 0)),
            pl.BlockSpec((ROWW, _LANES), lambda i: (0, 0)),
        ],
        out_specs=pl.BlockSpec((TB, _LANES), lambda i: (i, 0)),
        compiler_params=pltpu.CompilerParams(
            dimension_semantics=("parallel",),
            vmem_limit_bytes=48 * 1024 * 1024),
        cost_estimate=pl.CostEstimate(
            flops=2 * R * ROWW * _LANES,
            transcendentals=2 * R * ROWW + 2 * NC,
            bytes_accessed=R * ROWW * 4 + NC * 4),
    )(p_arr, x2, w)

    return out.astype(x.dtype).reshape(N, C, 1, 1)


# speedup vs baseline: 27.2647x; 27.2647x over previous
import functools

import jax
import jax.numpy as jnp
from jax.experimental import pallas as pl
from jax.experimental.pallas import tpu as pltpu


def _gem_body(p_ref, x_ref, o_ref, *, hw, eps):
    p = p_ref[0, 0]
    xc = jnp.maximum(x_ref[...], eps)
    xp = jnp.exp(jnp.log(xc) * p)
    s = jnp.sum(xp, axis=0)
    m = s * (1.0 / hw)
    o_ref[...] = jnp.exp(jnp.log(m) / p)


def kernel(x, p):
    N, C, H, W = x.shape
    HW = H * W
    xt = jnp.transpose(x, (2, 3, 0, 1)).reshape(HW, N, C)

    p_arr = jnp.asarray(p, jnp.float32).reshape(1, 1)

    BN = 16
    while N % BN:
        BN //= 2

    out = pl.pallas_call(
        functools.partial(_gem_body, hw=HW, eps=1e-6),
        out_shape=jax.ShapeDtypeStruct((N, C), x.dtype),
        grid=(N // BN,),
        in_specs=[
            pl.BlockSpec(memory_space=pltpu.MemorySpace.SMEM),
            pl.BlockSpec((HW, BN, C), lambda i: (0, i, 0)),
        ],
        out_specs=pl.BlockSpec((BN, C), lambda i: (i, 0)),
        compiler_params=pltpu.CompilerParams(
            dimension_semantics=("parallel",),
            vmem_limit_bytes=48 * 1024 * 1024),
        cost_estimate=pl.CostEstimate(
            flops=6 * N * C * HW,
            transcendentals=2 * N * C * HW + 2 * N * C,
            bytes_accessed=N * C * HW * 4 + N * C * 4),
    )(p_arr, xt)

    return out.reshape(N, C, 1, 1)
